# baseline (speedup 1.0000x reference)
import functools

import jax
import jax.numpy as jnp
from jax import lax
from jax.experimental import pallas as pl
from jax.experimental.pallas import tpu as pltpu
from jax.experimental.pallas import tpu_sc as plsc

B = 16384
F = 26
D = 32
N = 100000 * F
BF = B * F
GW = 128
NROWS = BF // GW
NW = 32
ROWS_PER_W = NROWS // NW
CHUNK_ROWS = 8
NCHUNK = ROWS_PER_W // CHUNK_ROWS
CHUNK_IDX = CHUNK_ROWS * GW
LANES = 16
OFF_PERIOD = 13

TILE_C = 128
FULL_TCOLS = N // TILE_C
TAIL_C = N - FULL_TCOLS * TILE_C
BT = 4
BC = BT * TILE_C
NB = 158
W_BIG = 24


TCB = 8192
TC_GRID = (N + TCB - 1) // TCB


def _tc_detile_transpose(table_t):
    m4 = TCB // 4

    def body(in_ref, out_ref):
        x = in_ref[...]
        x3 = x.reshape(D, m4, 4)
        y = jnp.transpose(x3, (1, 0, 2))
        y = jnp.transpose(y, (0, 2, 1))
        out_ref[...] = y.reshape(m4, 128)

    return pl.pallas_call(
        body,
        grid=(TC_GRID,),
        in_specs=[pl.BlockSpec((D, TCB), lambda i: (0, i))],
        out_specs=pl.BlockSpec((m4, 128), lambda i: (i, 0)),
        out_shape=jax.ShapeDtypeStruct((N * D // 128, 128), jnp.float32),
    )(table_t)


def _detile_transpose(table_t):
    mesh = plsc.VectorSubcoreMesh(core_axis_name="c", subcore_axis_name="s")

    @functools.partial(
        pl.kernel,
        mesh=mesh,
        out_type=jax.ShapeDtypeStruct((N * D,), jnp.float32),
        scratch_types=[
            pltpu.VMEM((D, BC), jnp.float32),
            pltpu.VMEM((D, BC), jnp.float32),
            pltpu.VMEM((BC * D,), jnp.float32),
            pltpu.VMEM((BC * D,), jnp.float32),
            pltpu.VMEM((D, TILE_C), jnp.float32),
            pltpu.VMEM((TILE_C * D,), jnp.float32),
            pltpu.VMEM((D, TAIL_C), jnp.float32),
            pltpu.VMEM((TAIL_C * D,), jnp.float32),
            pltpu.SemaphoreType.DMA,
            pltpu.SemaphoreType.DMA,
            pltpu.SemaphoreType.DMA,
            pltpu.SemaphoreType.DMA,
        ],
        compiler_params=pltpu.CompilerParams(needs_layout_passes=False),
    )
    def ka(tt_hbm, tlin_hbm, in_a, in_b, out_a, out_b, in_r, out_r,
           in_t, out_t, si_a, si_b, so_a, so_b):
        cid = lax.axis_index("c")
        sid = lax.axis_index("s")
        wid = sid * 2 + cid
        big = jnp.minimum(wid, W_BIG)
        n_t = jnp.where(wid < W_BIG, 635, 634)
        col0 = (big * 635 + jnp.maximum(wid - W_BIG, 0) * 634) * TILE_C

        lane = lax.iota(jnp.int32, LANES)
        idx_d = [lane * D + d for d in range(D)]

        def transpose(in_v, out_v, width):
            @plsc.parallel_loop(0, width // LANES, unroll=4)
            def body(q):
                p0 = q * LANES
                out_sl = out_v.at[pl.ds(q * (LANES * D), LANES * D)]
                for d in range(D):
                    v = in_v[d, pl.ds(p0, LANES)]
                    plsc.store_scatter(out_sl, [idx_d[d]], v)

        def fire_in(blk, buf, sem):
            return pltpu.async_copy(
                tt_hbm.at[:, pl.ds(col0 + blk * BC, BC)], buf, sem)

        def fire_out(blk, buf, sem):
            return pltpu.async_copy(
                buf, tlin_hbm.at[pl.ds((col0 + blk * BC) * D, BC * D)], sem)

        def drain_in(buf, sem):
            pltpu.make_async_copy(tt_hbm.at[:, pl.ds(0, BC)], buf, sem).wait()

        def drain_out(buf, sem):
            pltpu.make_async_copy(
                buf, tlin_hbm.at[pl.ds(0, BC * D)], sem).wait()

        fire_in(0, in_a, si_a)
        fire_in(1, in_b, si_b)

        def step(j, carry):
            blk = j * 2

            drain_in(in_a, si_a)
            @pl.when(j > 0)
            def _():
                drain_out(out_a, so_a)
            transpose(in_a, out_a, BC)

            @pl.when(blk + 2 < NB)
            def _():
                fire_in(blk + 2, in_a, si_a)
            fire_out(blk, out_a, so_a)

            drain_in(in_b, si_b)
            @pl.when(j > 0)
            def _():
                drain_out(out_b, so_b)
            transpose(in_b, out_b, BC)

            @pl.when(blk + 3 < NB)
            def _():
                fire_in(blk + 3, in_b, si_b)
            fire_out(blk + 1, out_b, so_b)
            return carry

        lax.fori_loop(0, NB // 2, step, 0)
        drain_out(out_a, so_a)
        drain_out(out_b, so_b)

        rem = n_t - NB * BT

        def rem_step(t, carry):
            c = col0 + NB * BC + t * TILE_C
            pltpu.async_copy(
                tt_hbm.at[:, pl.ds(c, TILE_C)], in_r, si_a).wait()
            transpose(in_r, out_r, TILE_C)
            pltpu.async_copy(
                out_r, tlin_hbm.at[pl.ds(c * D, TILE_C * D)], so_a).wait()
            return carry

        lax.fori_loop(0, rem, rem_step, 0)

        @pl.when(wid == NW - 1)
        def _():
            c = FULL_TCOLS * TILE_C
            pltpu.async_copy(
                tt_hbm.at[:, pl.ds(c, TAIL_C)], in_t, si_b).wait()
            transpose(in_t, out_t, TAIL_C)
            pltpu.async_copy(
                out_t, tlin_hbm.at[pl.ds(c * D, TAIL_C * D)], so_b).wait()

    return ka(table_t)


def _emb_lookup(ids2d, offs13, table_lin):
    mesh = plsc.VectorSubcoreMesh(core_axis_name="c", subcore_axis_name="s")

    @functools.partial(
        pl.kernel,
        mesh=mesh,
        out_type=jax.ShapeDtypeStruct((BF, D), jnp.float32),
        scratch_types=[
            pltpu.VMEM((ROWS_PER_W, GW), jnp.int32),
            pltpu.VMEM((OFF_PERIOD, GW), jnp.int32),
            pltpu.VMEM((CHUNK_IDX, D), jnp.float32),
            pltpu.SemaphoreType.DMA,
        ],
        compiler_params=pltpu.CompilerParams(use_tc_tiling_on_sc=False),
    )
    def k(ids_hbm, off_hbm, table_hbm, out_hbm, idx_v, off_v, buf, sem):
        cid = lax.axis_index("c")
        sid = lax.axis_index("s")
        wid = sid * 2 + cid
        row0 = wid * ROWS_PER_W

        pltpu.sync_copy(off_hbm, off_v)
        pltpu.sync_copy(ids_hbm.at[pl.ds(row0 * 1, ROWS_PER_W)], idx_v)

        def add_row(r, carry):
            phase = lax.rem(row0 + r, OFF_PERIOD)
            for c in range(GW // LANES):
                off = off_v[phase, pl.ds(c * LANES, LANES)]
                cur = idx_v[r, pl.ds(c * LANES, LANES)]
                idx_v[r, pl.ds(c * LANES, LANES)] = cur + off
            return carry

        lax.fori_loop(0, ROWS_PER_W, add_row, 0)

        def chunk(t, carry):
            copies = []
            for j in range(CHUNK_ROWS):
                r = t * CHUNK_ROWS + j
                cp = pltpu.async_copy(
                    table_hbm.at[idx_v.at[r]],
                    buf.at[pl.ds(j * GW, GW)],
                    sem,
                )
                copies.append(cp)
            for cp in copies:
                cp.wait()
            pltpu.sync_copy(
                buf, out_hbm.at[pl.ds(row0 * GW + t * CHUNK_IDX, CHUNK_IDX)]
            )
            return carry

        lax.fori_loop(0, NCHUNK, chunk, 0)

    return k(ids2d, offs13, table_lin)


def kernel(input_ids, offsets, table):
    ids2d = input_ids.reshape(NROWS, GW)
    offs13 = jnp.tile(offsets, OFF_PERIOD * GW // F).reshape(OFF_PERIOD, GW)
    tlin = _detile_transpose(table.T)
    out = _emb_lookup(ids2d, offs13, tlin.reshape(N, D))
    return out.reshape(B, F, D)


# speedup vs baseline: 1.6340x; 1.6340x over previous
import functools

import jax
import jax.numpy as jnp
from jax import lax
from jax.experimental import pallas as pl
from jax.experimental.pallas import tpu as pltpu
from jax.experimental.pallas import tpu_sc as plsc

B = 16384
F = 26
D = 32
N = 100000 * F
BF = B * F
GW = 128
NW = 32
LANES = 16

TILE_C = 128
FULL_TCOLS = N // TILE_C
TAIL_C = N - FULL_TCOLS * TILE_C
BT = 4
BC = BT * TILE_C
NB = 158
W_BIG = 24

NG = F * (B // GW)
G_PER_W = NG // NW


def _detile_dmajor(table_t):
    mesh = plsc.VectorSubcoreMesh(core_axis_name="c", subcore_axis_name="s")

    @functools.partial(
        pl.kernel,
        mesh=mesh,
        out_type=jax.ShapeDtypeStruct((N * D,), jnp.float32),
        scratch_types=[
            pltpu.VMEM((D, BC), jnp.float32),
            pltpu.VMEM((D, BC), jnp.float32),
            pltpu.VMEM((BC * D,), jnp.float32),
            pltpu.VMEM((BC * D,), jnp.float32),
            pltpu.VMEM((D, TILE_C), jnp.float32),
            pltpu.VMEM((TILE_C * D,), jnp.float32),
            pltpu.VMEM((D, TAIL_C), jnp.float32),
            pltpu.VMEM((TAIL_C * D,), jnp.float32),
            pltpu.SemaphoreType.DMA,
            pltpu.SemaphoreType.DMA,
            pltpu.SemaphoreType.DMA,
            pltpu.SemaphoreType.DMA,
        ],
        compiler_params=pltpu.CompilerParams(needs_layout_passes=False),
    )
    def ka(tt_hbm, tlin_hbm, in_a, in_b, out_a, out_b, in_r, out_r,
           in_t, out_t, si_a, si_b, so_a, so_b):
        cid = lax.axis_index("c")
        sid = lax.axis_index("s")
        wid = sid * 2 + cid
        big = jnp.minimum(wid, W_BIG)
        n_t = jnp.where(wid < W_BIG, 635, 634)
        col0 = (big * 635 + jnp.maximum(wid - W_BIG, 0) * 634) * TILE_C

        def detile(in_v, out_v, width):
            @plsc.parallel_loop(0, width // LANES, unroll=2)
            def body(q):
                p0 = q * LANES
                for d in range(D):
                    out_v[pl.ds(d * width + p0, LANES)] = (
                        in_v[d, pl.ds(p0, LANES)])

        def fire_in(blk, buf, sem):
            return pltpu.async_copy(
                tt_hbm.at[:, pl.ds(col0 + blk * BC, BC)], buf, sem)

        def drain_in(buf, sem):
            pltpu.make_async_copy(
                tt_hbm.at[:, pl.ds(0, BC)], buf, sem).wait()

        def fire_outs(c, out_v, width, sem):
            for d in range(D):
                pltpu.async_copy(
                    out_v.at[pl.ds(d * width, width)],
                    tlin_hbm.at[pl.ds(d * N + c, width)], sem)

        def drain_outs(out_v, width, sem):
            for d in range(D):
                pltpu.make_async_copy(
                    out_v.at[pl.ds(d * width, width)],
                    tlin_hbm.at[pl.ds(0, width)], sem).wait()

        fire_in(0, in_a, si_a)
        fire_in(1, in_b, si_b)

        def step(j, carry):
            blk = j * 2
            drain_in(in_a, si_a)

            @pl.when(j > 0)
            def _():
                drain_outs(out_a, BC, so_a)
            detile(in_a, out_a, BC)

            @pl.when(blk + 2 < NB)
            def _():
                fire_in(blk + 2, in_a, si_a)
            fire_outs(col0 + blk * BC, out_a, BC, so_a)

            drain_in(in_b, si_b)

            @pl.when(j > 0)
            def _():
                drain_outs(out_b, BC, so_b)
            detile(in_b, out_b, BC)

            @pl.when(blk + 3 < NB)
            def _():
                fire_in(blk + 3, in_b, si_b)
            fire_outs(col0 + (blk + 1) * BC, out_b, BC, so_b)
            return carry

        lax.fori_loop(0, NB // 2, step, 0)
        drain_outs(out_a, BC, so_a)
        drain_outs(out_b, BC, so_b)

        rem = n_t - NB * BT

        def rem_step(t, carry):
            c = col0 + NB * BC + t * TILE_C
            pltpu.async_copy(
                tt_hbm.at[:, pl.ds(c, TILE_C)], in_r, si_a).wait()
            detile(in_r, out_r, TILE_C)
            fire_outs(c, out_r, TILE_C, so_a)
            drain_outs(out_r, TILE_C, so_a)
            return carry

        lax.fori_loop(0, rem, rem_step, 0)

        @pl.when(wid == NW - 1)
        def _():
            ct = FULL_TCOLS * TILE_C
            pltpu.async_copy(
                tt_hbm.at[:, pl.ds(ct, TAIL_C)], in_t, si_b).wait()
            detile(in_t, out_t, TAIL_C)
            fire_outs(ct, out_t, TAIL_C, so_b)
            drain_outs(out_t, TAIL_C, so_b)

    return ka(table_t)


def _emb_lookup_dmajor(ids_t, offsets_pad, table_flat):
    mesh = plsc.VectorSubcoreMesh(core_axis_name="c", subcore_axis_name="s")

    @functools.partial(
        pl.kernel,
        mesh=mesh,
        out_type=jax.ShapeDtypeStruct((BF * D,), jnp.float32),
        scratch_types=[
            pltpu.VMEM((GW,), jnp.int32),
            pltpu.VMEM((32,), jnp.int32),
            pltpu.VMEM((D, GW), jnp.int32),
            pltpu.VMEM((D * GW,), jnp.float32),
            pltpu.SemaphoreType.DMA,
        ],
        compiler_params=pltpu.CompilerParams(
            use_tc_tiling_on_sc=False, needs_layout_passes=False),
    )
    def kb(ids_hbm, off_hbm, t_hbm, out_hbm, iv, off_v, idxb, data, sem):
        cid = lax.axis_index("c")
        sid = lax.axis_index("s")
        wid = sid * 2 + cid
        g0 = wid * G_PER_W

        pltpu.sync_copy(off_hbm, off_v)

        def g_step(t, carry):
            g = g0 + t
            f = g // (B // GW)
            bh = g % (B // GW)
            pltpu.sync_copy(ids_hbm.at[f, pl.ds(bh * GW, GW)], iv)
            fv = jnp.full((LANES,), f, jnp.int32)
            offv = plsc.load_gather(off_v, [fv])
            for k in range(GW // LANES):
                i = iv[pl.ds(k * LANES, LANES)] + offv
                for d in range(D):
                    idxb[d, pl.ds(k * LANES, LANES)] = i + (d * N)
            copies = []
            for d in range(D):
                copies.append(pltpu.async_copy(
                    t_hbm.at[idxb.at[d]], data.at[pl.ds(d * GW, GW)], sem))
            for cp in copies:
                cp.wait()
            for dh in range(D // 8):
                base = (f * 512 + dh * 128 + bh) * 1024
                pltpu.sync_copy(
                    data.at[pl.ds(dh * 1024, 1024)],
                    out_hbm.at[pl.ds(base, 1024)])
            return carry

        lax.fori_loop(0, G_PER_W, g_step, 0)

    return kb(ids_t, offsets_pad, table_flat)


def kernel(input_ids, offsets, table):
    ids_t = input_ids.T
    offsets_pad = jnp.concatenate(
        [offsets, jnp.zeros((32 - F,), jnp.int32)])
    tlin = _detile_dmajor(table.T)
    out1 = _emb_lookup_dmajor(ids_t, offsets_pad, tlin)
    out5 = out1.reshape(F, D // 8, B // GW, 8, GW)
    return jnp.transpose(out5, (2, 4, 0, 1, 3)).reshape(B, F, D)


# speedup vs baseline: 1.9902x; 1.2180x over previous
import functools

import jax
import jax.numpy as jnp
from jax import lax
from jax.experimental import pallas as pl
from jax.experimental.pallas import tpu as pltpu
from jax.experimental.pallas import tpu_sc as plsc

B = 16384
F = 26
D = 32
N = 100000 * F
BF = B * F
GW = 128
NW = 32
LANES = 16

TILE_C = 128
FULL_TCOLS = N // TILE_C
TAIL_C = N - FULL_TCOLS * TILE_C
BT = 4
BC = BT * TILE_C
NB = 158
W_BIG = 24

NG = F * (B // GW)
G_PER_W = NG // NW


def _detile_dmajor(table_t):
    mesh = plsc.VectorSubcoreMesh(core_axis_name="c", subcore_axis_name="s")

    @functools.partial(
        pl.kernel,
        mesh=mesh,
        out_type=jax.ShapeDtypeStruct((N * D,), jnp.float32),
        scratch_types=[
            pltpu.VMEM((D, BC), jnp.float32),
            pltpu.VMEM((D, BC), jnp.float32),
            pltpu.VMEM((BC * D,), jnp.float32),
            pltpu.VMEM((BC * D,), jnp.float32),
            pltpu.VMEM((D, TILE_C), jnp.float32),
            pltpu.VMEM((TILE_C * D,), jnp.float32),
            pltpu.VMEM((D, TAIL_C), jnp.float32),
            pltpu.VMEM((TAIL_C * D,), jnp.float32),
            pltpu.SemaphoreType.DMA,
            pltpu.SemaphoreType.DMA,
            pltpu.SemaphoreType.DMA,
            pltpu.SemaphoreType.DMA,
        ],
        compiler_params=pltpu.CompilerParams(needs_layout_passes=False),
    )
    def ka(tt_hbm, tlin_hbm, in_a, in_b, out_a, out_b, in_r, out_r,
           in_t, out_t, si_a, si_b, so_a, so_b):
        cid = lax.axis_index("c")
        sid = lax.axis_index("s")
        wid = sid * 2 + cid
        big = jnp.minimum(wid, W_BIG)
        n_t = jnp.where(wid < W_BIG, 635, 634)
        col0 = (big * 635 + jnp.maximum(wid - W_BIG, 0) * 634) * TILE_C

        def detile(in_v, out_v, width):
            @plsc.parallel_loop(0, width // LANES, unroll=2)
            def body(q):
                p0 = q * LANES
                for d in range(D):
                    out_v[pl.ds(d * width + p0, LANES)] = (
                        in_v[d, pl.ds(p0, LANES)])

        def fire_in(blk, buf, sem):
            return pltpu.async_copy(
                tt_hbm.at[:, pl.ds(col0 + blk * BC, BC)], buf, sem)

        def drain_in(buf, sem):
            pltpu.make_async_copy(
                tt_hbm.at[:, pl.ds(0, BC)], buf, sem).wait()

        def fire_outs(c, out_v, width, sem):
            for d in range(D):
                pltpu.async_copy(
                    out_v.at[pl.ds(d * width, width)],
                    tlin_hbm.at[pl.ds(d * N + c, width)], sem)

        def drain_outs(out_v, width, sem):
            for d in range(D):
                pltpu.make_async_copy(
                    out_v.at[pl.ds(d * width, width)],
                    tlin_hbm.at[pl.ds(0, width)], sem).wait()

        fire_in(0, in_a, si_a)
        fire_in(1, in_b, si_b)

        def step(j, carry):
            blk = j * 2
            drain_in(in_a, si_a)

            @pl.when(j > 0)
            def _():
                drain_outs(out_a, BC, so_a)
            detile(in_a, out_a, BC)

            @pl.when(blk + 2 < NB)
            def _():
                fire_in(blk + 2, in_a, si_a)
            fire_outs(col0 + blk * BC, out_a, BC, so_a)

            drain_in(in_b, si_b)

            @pl.when(j > 0)
            def _():
                drain_outs(out_b, BC, so_b)
            detile(in_b, out_b, BC)

            @pl.when(blk + 3 < NB)
            def _():
                fire_in(blk + 3, in_b, si_b)
            fire_outs(col0 + (blk + 1) * BC, out_b, BC, so_b)
            return carry

        lax.fori_loop(0, NB // 2, step, 0)
        drain_outs(out_a, BC, so_a)
        drain_outs(out_b, BC, so_b)

        rem = n_t - NB * BT

        def rem_step(t, carry):
            c = col0 + NB * BC + t * TILE_C
            pltpu.async_copy(
                tt_hbm.at[:, pl.ds(c, TILE_C)], in_r, si_a).wait()
            detile(in_r, out_r, TILE_C)
            fire_outs(c, out_r, TILE_C, so_a)
            drain_outs(out_r, TILE_C, so_a)
            return carry

        lax.fori_loop(0, rem, rem_step, 0)

        @pl.when(wid == NW - 1)
        def _():
            ct = FULL_TCOLS * TILE_C
            pltpu.async_copy(
                tt_hbm.at[:, pl.ds(ct, TAIL_C)], in_t, si_b).wait()
            detile(in_t, out_t, TAIL_C)
            fire_outs(ct, out_t, TAIL_C, so_b)
            drain_outs(out_t, TAIL_C, so_b)

    return ka(table_t)


def _emb_lookup_dmajor(ids_t, offsets_pad, table_flat):
    mesh = plsc.VectorSubcoreMesh(core_axis_name="c", subcore_axis_name="s")

    @functools.partial(
        pl.kernel,
        mesh=mesh,
        out_type=jax.ShapeDtypeStruct((BF * D // GW, GW), jnp.float32),
        scratch_types=[
            pltpu.VMEM((GW,), jnp.int32),
            pltpu.VMEM((GW,), jnp.int32),
            pltpu.VMEM((32,), jnp.int32),
            pltpu.VMEM((D, GW), jnp.int32),
            pltpu.VMEM((D, GW), jnp.int32),
            pltpu.VMEM((D, GW), jnp.float32),
            pltpu.VMEM((D, GW), jnp.float32),
            pltpu.SemaphoreType.DMA,
            pltpu.SemaphoreType.DMA,
            pltpu.SemaphoreType.DMA,
            pltpu.SemaphoreType.DMA,
        ],
        compiler_params=pltpu.CompilerParams(
            use_tc_tiling_on_sc=False, needs_layout_passes=False),
    )
    def kb(ids_hbm, off_hbm, t_hbm, out_hbm, iv_a, iv_b, off_v,
           idx_a, idx_b, dat_a, dat_b, sg_a, sg_b, so_a, so_b):
        cid = lax.axis_index("c")
        sid = lax.axis_index("s")
        wid = sid * 2 + cid
        g0 = wid * G_PER_W
        nj = G_PER_W // 2

        pltpu.sync_copy(off_hbm, off_v)

        def prep(g, iv, idxb):
            f = g // (B // GW)
            bh = g % (B // GW)
            pltpu.sync_copy(ids_hbm.at[f, pl.ds(bh * GW, GW)], iv)
            fv = jnp.full((LANES,), f, jnp.int32)
            offv = plsc.load_gather(off_v, [fv])
            for k in range(GW // LANES):
                i = iv[pl.ds(k * LANES, LANES)] + offv
                for d in range(D):
                    idxb[d, pl.ds(k * LANES, LANES)] = i + (d * N)

        def fire_g(idxb, dat, sem):
            for d in range(D):
                pltpu.async_copy(t_hbm.at[idxb.at[d]], dat.at[d], sem)

        def drain_g(dat, sem):
            for d in range(D):
                pltpu.make_async_copy(
                    t_hbm.at[pl.ds(0, GW)], dat.at[d], sem).wait()

        def fire_outs(g, dat, sem):
            f = g // (B // GW)
            bh = g % (B // GW)
            for dh in range(D // 8):
                row = (f * 512 + dh * 128 + bh) * 8
                pltpu.async_copy(
                    dat.at[pl.ds(dh * 8, 8)], out_hbm.at[pl.ds(row, 8)], sem)

        def drain_outs(dat, sem):
            for dh in range(D // 8):
                pltpu.make_async_copy(
                    dat.at[pl.ds(dh * 8, 8)], out_hbm.at[pl.ds(0, 8)],
                    sem).wait()

        prep(g0, iv_a, idx_a)
        fire_g(idx_a, dat_a, sg_a)

        def step(j, carry):
            ga = g0 + 2 * j
            gb = ga + 1

            prep(gb, iv_b, idx_b)

            @pl.when(j > 0)
            def _():
                drain_outs(dat_b, so_b)
            fire_g(idx_b, dat_b, sg_b)

            drain_g(dat_a, sg_a)
            fire_outs(ga, dat_a, so_a)

            @pl.when(j + 1 < nj)
            def _():
                prep(ga + 2, iv_a, idx_a)
                drain_outs(dat_a, so_a)
                fire_g(idx_a, dat_a, sg_a)

            drain_g(dat_b, sg_b)
            fire_outs(gb, dat_b, so_b)
            return carry

        lax.fori_loop(0, nj, step, 0)
        drain_outs(dat_a, so_a)
        drain_outs(dat_b, so_b)

    return kb(ids_t, offsets_pad, table_flat)


def kernel(input_ids, offsets, table):
    ids_t = input_ids.T
    offsets_pad = jnp.concatenate(
        [offsets, jnp.zeros((32 - F,), jnp.int32)])
    tlin = _detile_dmajor(table.T)
    out2 = _emb_lookup_dmajor(ids_t, offsets_pad, tlin)
    out5 = out2.reshape(F, D // 8, B // GW, 8, GW)
    return jnp.transpose(out5, (2, 4, 0, 1, 3)).reshape(B, F, D)
